# Initial kernel scaffold; baseline (speedup 1.0000x reference)
import jax
import jax.numpy as jnp
from jax.experimental import pallas as pl


def kernel(x, adj, tra1, tra2, tra3, z, W1, W2, W3, W4, W5):
    raise NotImplementedError("write your pallas kernel here")


# speedup vs baseline: 3.6270x; 3.6270x over previous
import jax
import jax.numpy as jnp
from jax import lax
from jax.experimental import pallas as pl
from jax.experimental.pallas import tpu as pltpu
from jax.experimental.pallas import tpu_sc as plsc

N = 10000
NS = 16
NPAD = 10240
RPT = NPAD // NS
CH = 128
D = 128
E = 320000
EPAD = 323584
EH = EPAD // 2
E_TILE_COL = EPAD // NS
E_TILE_EDG = EH // NS
RB = 400
SIG = 0.5


def _seg_sum_sc(table, src, dst, column_split):
    ept = E_TILE_COL if column_split else E_TILE_EDG
    nit = ept // CH
    nz = RPT // CH
    mesh = plsc.VectorSubcoreMesh(core_axis_name="c", subcore_axis_name="s")

    def body(table_r, src_r, dst_r, out_r, acc, idx_s, idx_d, rows, sem):
        c = lax.axis_index("c")
        s = lax.axis_index("s")

        zero16 = jnp.zeros((16,), jnp.float32)

        def zrow(i, _):
            for k in range(D // 16):
                rows[i, pl.ds(k * 16, 16)] = zero16
            return 0

        lax.fori_loop(0, CH, zrow, 0)
        for j in range(nz):
            pltpu.sync_copy(rows, acc.at[pl.ds(s * RPT + j * CH, CH)])
        plsc.subcore_barrier()

        def step(i, _):
            if column_split:
                s0 = c * EPAD + s * ept + i * CH
                d0 = s * ept + i * CH
            else:
                s0 = c * EH + s * ept + i * CH
                d0 = s0
            pltpu.sync_copy(src_r.at[pl.ds(s0, CH)], idx_s)
            pltpu.sync_copy(dst_r.at[pl.ds(d0, CH)], idx_d)
            pltpu.async_copy(table_r.at[idx_s], rows, sem).wait()
            pltpu.sync_copy(rows, acc.at[idx_d], add=True)
            return 0

        lax.fori_loop(0, nit, step, 0)
        plsc.subcore_barrier()

        for j in range(nz):
            r0 = s * RPT + j * CH
            pltpu.sync_copy(acc.at[pl.ds(r0, CH)], rows)
            pltpu.sync_copy(rows, out_r.at[pl.ds(c * NPAD + r0, CH)])

    f = pl.kernel(
        body,
        out_type=jax.ShapeDtypeStruct((2 * NPAD, D), jnp.float32),
        mesh=mesh,
        scratch_types=[
            pltpu.VMEM_SHARED((NPAD, D), jnp.float32),
            pltpu.VMEM((CH,), jnp.int32),
            pltpu.VMEM((CH,), jnp.int32),
            pltpu.VMEM((CH, D), jnp.float32),
            pltpu.SemaphoreType.DMA,
        ],
    )
    return f(table, src, dst).reshape(2, NPAD, D)


def _spec_part(part):
    return pl.BlockSpec((1, RB, D), lambda i, _p=part: (_p, i, 0))


def _tc_layer1(g0, tra1, W1, W2):
    def body(a, b, tr, w1, w2, out):
        g = a[0] + b[0]
        t = jnp.dot(g, w1[...], preferred_element_type=jnp.float32)
        h = SIG * (jnp.maximum(t, 0.0) + tr[...])
        s2 = jnp.dot(h, w2[...], preferred_element_type=jnp.float32)
        out[0] = s2[:, 0:128]
        out[1] = s2[:, 128:256]

    return pl.pallas_call(
        body,
        grid=(N // RB,),
        in_specs=[
            _spec_part(0),
            _spec_part(1),
            pl.BlockSpec((RB, 256), lambda i: (i, 0)),
            pl.BlockSpec((128, 256), lambda i: (0, 0)),
            pl.BlockSpec((256, 256), lambda i: (0, 0)),
        ],
        out_specs=pl.BlockSpec((2, RB, 128), lambda i: (0, i, 0)),
        out_shape=jax.ShapeDtypeStruct((2, N, 128), jnp.float32),
    )(g0, g0, tra1, W1, W2)


def _tc_layer2(o2, tra2, W3):
    def body(a, b, tr, w, out):
        ha = SIG * (jnp.maximum(a[0], 0.0) + tr[:, 0:128])
        hb = SIG * (jnp.maximum(b[0], 0.0) + tr[:, 128:256])
        s = jnp.dot(ha, w[0:128, :], preferred_element_type=jnp.float32)
        s = s + jnp.dot(hb, w[128:256, :], preferred_element_type=jnp.float32)
        out[...] = s

    return pl.pallas_call(
        body,
        grid=(N // RB,),
        in_specs=[
            _spec_part(0),
            _spec_part(1),
            pl.BlockSpec((RB, 256), lambda i: (i, 0)),
            pl.BlockSpec((256, 128), lambda i: (0, 0)),
        ],
        out_specs=pl.BlockSpec((RB, 128), lambda i: (i, 0)),
        out_shape=jax.ShapeDtypeStruct((N, 128), jnp.float32),
    )(o2, o2, tra2, W3)


def _tc_layer3(o3, tra3, W4):
    def body(a, b, tr, w, out):
        g = a[0] + b[0]
        h = SIG * (jnp.maximum(g, 0.0) + tr[...])
        s = jnp.dot(h, w[...], preferred_element_type=jnp.float32)
        out[...] = jnp.zeros((RB, D), jnp.float32)
        out[:, 0:32] = s

    return pl.pallas_call(
        body,
        grid=(N // RB,),
        in_specs=[
            _spec_part(0),
            _spec_part(1),
            pl.BlockSpec((RB, 128), lambda i: (i, 0)),
            pl.BlockSpec((128, 32), lambda i: (0, 0)),
        ],
        out_specs=pl.BlockSpec((RB, D), lambda i: (i, 0)),
        out_shape=jax.ShapeDtypeStruct((N, D), jnp.float32),
    )(o3, o3, tra3, W4)


def _tc_layer4(o4, z):
    def body(a, b, zr, out):
        g = a[0] + b[0]
        h = SIG * (jnp.maximum(g[:, 0:32], 0.0) + zr[...])
        out[...] = jnp.zeros((RB, D), jnp.float32)
        out[:, 0:32] = h

    return pl.pallas_call(
        body,
        grid=(N // RB,),
        in_specs=[
            _spec_part(0),
            _spec_part(1),
            pl.BlockSpec((RB, 32), lambda i: (i, 0)),
        ],
        out_specs=pl.BlockSpec((RB, D), lambda i: (i, 0)),
        out_shape=jax.ShapeDtypeStruct((N, D), jnp.float32),
    )(o4, o4, z)


def _tc_final(g5, W5):
    def body(a, b, w, out):
        g = a[0] + b[0]
        s = jnp.dot(g[:, 0:32], w[...], preferred_element_type=jnp.float32)
        out[...] = s

    return pl.pallas_call(
        body,
        grid=(N // RB,),
        in_specs=[
            _spec_part(0),
            _spec_part(1),
            pl.BlockSpec((32, 16), lambda i: (0, 0)),
        ],
        out_specs=pl.BlockSpec((RB, 16), lambda i: (i, 0)),
        out_shape=jax.ShapeDtypeStruct((N, 16), jnp.float32),
    )(g5, g5, W5)


def kernel(x, adj, tra1, tra2, tra3, z, W1, W2, W3, W4, W5):
    src = adj[0]
    dst = adj[1]
    pad_e = EPAD - E
    srcp = jnp.concatenate([src, jnp.zeros((pad_e,), jnp.int32)])
    dstp = jnp.concatenate([dst, jnp.full((pad_e,), NPAD - 1, jnp.int32)])
    srcb = jnp.concatenate([srcp, srcp + N])

    g0 = _seg_sum_sc(x, srcp, dstp, column_split=False)
    s2 = _tc_layer1(g0, tra1, W1, W2)
    o2 = _seg_sum_sc(s2.reshape(2 * N, D), srcb, dstp, column_split=True)
    s3 = _tc_layer2(o2, tra2, W3)
    o3 = _seg_sum_sc(s3, srcp, dstp, column_split=False)
    s4 = _tc_layer3(o3, tra3, W4)
    o4 = _seg_sum_sc(s4, srcp, dstp, column_split=False)
    h5 = _tc_layer4(o4, z)
    g5 = _seg_sum_sc(h5, srcp, dstp, column_split=False)
    return _tc_final(g5, W5)
